# baseline (speedup 1.0000x reference)
import jax
import jax.numpy as jnp
from jax import lax
from jax.experimental import pallas as pl
from jax.experimental.pallas import tpu as pltpu
from jax.experimental.pallas import tpu_sc as plsc

_NC = 2
_NS = 16
_NW = _NC * _NS
_L = 16


def _sc_body(table_hbm, idx_hbm, out_hbm, table_v, idx_v, out_v):
    n = idx_v.shape[0]
    wid = lax.axis_index("s") * _NC + lax.axis_index("c")
    base = wid * n
    pltpu.sync_copy(table_hbm, table_v)
    pltpu.sync_copy(idx_hbm.at[pl.ds(base, n)], idx_v)

    t0 = table_v[pl.ds(0, _L)]
    t1 = table_v[pl.ds(_L, _L)]
    zero = jnp.zeros((_L,), jnp.int32)

    unroll = 16
    chunk = unroll * _L

    def step(i, carry):
        off = i * chunk
        for j in range(unroll):
            o = off + j * _L
            x = idx_v[pl.ds(o, _L)]
            out_v[pl.ds(o, _L)] = jnp.where(x == zero, t0, t1)
        return carry

    lax.fori_loop(0, n // chunk, step, 0)
    pltpu.sync_copy(out_v, out_hbm.at[pl.ds(base, n)])


def kernel(inputs, table):
    B, S = inputs.shape
    n_total = B * S
    per_w = n_total // _NW
    flat = inputs.reshape(n_total).astype(jnp.int32)
    t01 = jnp.repeat(table.reshape(-1).astype(jnp.float32), _L)
    mesh = plsc.VectorSubcoreMesh(core_axis_name="c", subcore_axis_name="s",
                                  num_cores=_NC, num_subcores=_NS)
    f = pl.kernel(
        _sc_body,
        out_type=jax.ShapeDtypeStruct((n_total,), jnp.float32),
        mesh=mesh,
        scratch_types=[
            pltpu.VMEM((2 * _L,), jnp.float32),
            pltpu.VMEM((per_w,), jnp.int32),
            pltpu.VMEM((per_w,), jnp.float32),
        ],
    )
    return f(t01, flat).reshape(B, S)


# speedup vs baseline: 1.1552x; 1.1552x over previous
import jax
import jax.numpy as jnp
from jax import lax
from jax.experimental import pallas as pl
from jax.experimental.pallas import tpu as pltpu
from jax.experimental.pallas import tpu_sc as plsc

_NC = 2
_NS = 16
_NW = _NC * _NS
_L = 16


_CHUNK = 128


def _sc_body(table_hbm, idx_hbm, out_hbm, table_v, idx_v, out_v):
    chunk, cols = idx_v.shape
    total_rows = idx_hbm.shape[0]
    rows_per_w = total_rows // _NW
    wid = lax.axis_index("s") * _NC + lax.axis_index("c")
    base = wid * rows_per_w
    pltpu.sync_copy(table_hbm, table_v)

    t0 = table_v[0, :]
    t1 = table_v[1, :]
    zero = jnp.zeros((_L,), jnp.int32)
    vregs_per_row = cols // _L

    def do_chunk(k, carry):
        r0 = base + k * chunk
        pltpu.sync_copy(idx_hbm.at[pl.ds(r0, chunk)], idx_v)
        for j in range(chunk):
            for c in range(vregs_per_row):
                x = idx_v[j, pl.ds(c * _L, _L)]
                out_v[j, pl.ds(c * _L, _L)] = jnp.where(x == zero, t0, t1)
        pltpu.sync_copy(out_v, out_hbm.at[pl.ds(r0, chunk)])
        return carry

    lax.fori_loop(0, rows_per_w // chunk, do_chunk, 0)


def kernel(inputs, table):
    B, S = inputs.shape
    idx = inputs.astype(jnp.int32)
    t01 = jnp.repeat(table.reshape(2, 1).astype(jnp.float32), _L, axis=1)
    mesh = plsc.VectorSubcoreMesh(core_axis_name="c", subcore_axis_name="s",
                                  num_cores=_NC, num_subcores=_NS)
    f = pl.kernel(
        _sc_body,
        out_type=jax.ShapeDtypeStruct((B, S), jnp.float32),
        mesh=mesh,
        scratch_types=[
            pltpu.VMEM((2, _L), jnp.float32),
            pltpu.VMEM((_CHUNK, S), jnp.int32),
            pltpu.VMEM((_CHUNK, S), jnp.float32),
        ],
    )
    return f(t01, idx)
